# baseline (device time: 48352 ns/iter reference)
import jax
import jax.numpy as jnp
from jax import lax
from jax.experimental import pallas as pl
from jax.experimental.pallas import tpu as pltpu

N_DEV = 4
B, SQ, D = 2, 256, 768
HQ_LOC, DH = 8, 64
KV_COLS = 2 * DH


def kernel(x, Wq, Wo, Wk, Wv):
    def body(x_ref, wq_ref, wo_ref, wk_ref, wv_ref, out_ref,
             comm_ref, acc_ref, send_sems, recv_sems):
        my = lax.axis_index("i")
        left = (my - 1) % N_DEV
        right = (my + 1) % N_DEV

        barrier_sem = pltpu.get_barrier_semaphore()
        for nbr in [left, right]:
            pl.semaphore_signal(
                barrier_sem, inc=1,
                device_id=(nbr,), device_id_type=pl.DeviceIdType.MESH,
            )
        pl.semaphore_wait(barrier_sem, 2)

        wq = wq_ref[...].astype(jnp.bfloat16)
        wk = wk_ref[:, pl.ds(my * KV_COLS, KV_COLS)].astype(jnp.bfloat16)
        wv = wv_ref[:, pl.ds(my * KV_COLS, KV_COLS)].astype(jnp.bfloat16)
        wo = wo_ref[...].astype(jnp.bfloat16)

        for b in range(B):
            xb = x_ref[b].astype(jnp.bfloat16)
            q = jnp.dot(xb, wq, preferred_element_type=jnp.float32
                        ).astype(jnp.bfloat16)
            k = jnp.dot(xb, wk, preferred_element_type=jnp.float32
                        ).astype(jnp.bfloat16)
            v = jnp.dot(xb, wv, preferred_element_type=jnp.float32
                        ).astype(jnp.bfloat16)

            for h in range(HQ_LOC):
                qh = q[:, h * DH:(h + 1) * DH]
                g = h // 4
                kh = k[:, g * DH:(g + 1) * DH]
                vh = v[:, g * DH:(g + 1) * DH]
                s = lax.dot_general(
                    qh, kh, (((1,), (1,)), ((), ())),
                    preferred_element_type=jnp.float32,
                ) * 0.125
                m = jnp.max(s, axis=1, keepdims=True)
                p = jnp.exp(s - m)
                l = jnp.sum(p, axis=1, keepdims=True)
                o = jnp.dot(p.astype(jnp.bfloat16), vh,
                            preferred_element_type=jnp.float32) / l
                acc_ref[:, h * DH:(h + 1) * DH] = o

            partial = jnp.dot(acc_ref[...].astype(jnp.bfloat16), wo,
                              preferred_element_type=jnp.float32)
            out_ref[b] = partial
            comm_ref[0, b] = partial.astype(jnp.bfloat16)

        for h in range(N_DEV - 1):
            rdma = pltpu.make_async_remote_copy(
                src_ref=comm_ref.at[h],
                dst_ref=comm_ref.at[h + 1],
                send_sem=send_sems.at[h],
                recv_sem=recv_sems.at[h],
                device_id=(right,),
                device_id_type=pl.DeviceIdType.MESH,
            )
            rdma.start()
            rdma.wait()
            out_ref[...] += comm_ref[h + 1][...].astype(jnp.float32)

    return pl.pallas_call(
        body,
        out_shape=jax.ShapeDtypeStruct((B, SQ, D), jnp.float32),
        in_specs=[pl.BlockSpec(memory_space=pltpu.VMEM)] * 5,
        out_specs=pl.BlockSpec(memory_space=pltpu.VMEM),
        scratch_shapes=[
            pltpu.VMEM((N_DEV, B, SQ, D), jnp.bfloat16),
            pltpu.VMEM((SQ, HQ_LOC * DH), jnp.float32),
            pltpu.SemaphoreType.DMA((N_DEV - 1,)),
            pltpu.SemaphoreType.DMA((N_DEV - 1,)),
        ],
        compiler_params=pltpu.CompilerParams(collective_id=0),
    )(x, Wq, Wo, Wk, Wv)


# device time: 28910 ns/iter; 1.6725x vs baseline; 1.6725x over previous
import jax
import jax.numpy as jnp
from jax import lax
from jax.experimental import pallas as pl
from jax.experimental.pallas import tpu as pltpu

N_DEV = 4
B, SQ, D = 2, 256, 768
T = B * SQ
QROWS = T // N_DEV
HQ_LOC, DH = 8, 64
HD_LOC = HQ_LOC * DH
KV_COLS = 2 * DH


def kernel(x, Wq, Wo, Wk, Wv):
    def body(x_ref, wq_ref, wo_ref, wk_ref, wv_ref, out_ref,
             stage_ref, rs_recv_ref, agq_ref, ag_recv_ref, acc_ref,
             rs_send_sems, rs_recv_sems, ag_send_sems, ag_recv_sems):
        my = lax.axis_index("i")

        barrier_sem = pltpu.get_barrier_semaphore()
        for j in range(1, N_DEV):
            pl.semaphore_signal(
                barrier_sem, inc=1,
                device_id=((my + j) % N_DEV,),
                device_id_type=pl.DeviceIdType.MESH,
            )
        pl.semaphore_wait(barrier_sem, N_DEV - 1)

        wqkv = jnp.concatenate(
            [wq_ref[...],
             wk_ref[:, pl.ds(my * KV_COLS, KV_COLS)],
             wv_ref[:, pl.ds(my * KV_COLS, KV_COLS)]],
            axis=1,
        ).astype(jnp.bfloat16)
        wo = wo_ref[...].astype(jnp.bfloat16)

        def rs_rdma(q):
            return pltpu.make_async_remote_copy(
                src_ref=stage_ref.at[q],
                dst_ref=rs_recv_ref.at[(my - q) % N_DEV - 1],
                send_sem=rs_send_sems.at[q],
                recv_sem=rs_recv_sems.at[(my - q) % N_DEV - 1],
                device_id=(q,),
                device_id_type=pl.DeviceIdType.MESH,
            )

        for b in range(B):
            xb = x_ref[b].astype(jnp.bfloat16)
            qkv = jnp.dot(xb, wqkv, preferred_element_type=jnp.float32
                          ).astype(jnp.bfloat16)
            q = qkv[:, :HD_LOC]
            k = qkv[:, HD_LOC:HD_LOC + KV_COLS]
            v = qkv[:, HD_LOC + KV_COLS:]

            for h in range(HQ_LOC):
                qh = q[:, h * DH:(h + 1) * DH]
                g = h // 4
                kh = k[:, g * DH:(g + 1) * DH]
                vh = v[:, g * DH:(g + 1) * DH]
                s = lax.dot_general(
                    qh, kh, (((1,), (1,)), ((), ())),
                    preferred_element_type=jnp.float32,
                ) * 0.125
                m = jnp.max(s, axis=1, keepdims=True)
                p = jnp.exp(s - m)
                l = jnp.sum(p, axis=1, keepdims=True)
                o = jnp.dot(p.astype(jnp.bfloat16), vh,
                            preferred_element_type=jnp.float32) / l
                acc_ref[:, h * DH:(h + 1) * DH] = o

            partial = jnp.dot(acc_ref[...].astype(jnp.bfloat16), wo,
                              preferred_element_type=jnp.float32)

            for half in range(2):
                qi = 2 * b + half
                stage_ref[qi] = partial[half * QROWS:(half + 1) * QROWS
                                        ].astype(jnp.bfloat16)

                @pl.when(qi != my)
                def _():
                    rs_rdma(qi).start()

        def recv_only(dst_ref, recv_sem):
            return pltpu.make_async_remote_copy(
                src_ref=dst_ref, dst_ref=dst_ref,
                send_sem=rs_send_sems.at[0], recv_sem=recv_sem,
                device_id=(my,), device_id_type=pl.DeviceIdType.MESH,
            )

        for j in range(N_DEV - 1):
            recv_only(rs_recv_ref.at[j], rs_recv_sems.at[j]).wait_recv()
        red = stage_ref[my].astype(jnp.float32)
        for j in range(N_DEV - 1):
            red += rs_recv_ref[j].astype(jnp.float32)
        out_ref[pl.ds(my * QROWS, QROWS), :] = red
        agq_ref[...] = red.astype(jnp.bfloat16)

        ag_sends = []
        for j in range(1, N_DEV):
            rdma = pltpu.make_async_remote_copy(
                src_ref=agq_ref,
                dst_ref=ag_recv_ref.at[N_DEV - 1 - j],
                send_sem=ag_send_sems.at[j - 1],
                recv_sem=ag_recv_sems.at[N_DEV - 1 - j],
                device_id=((my + j) % N_DEV,),
                device_id_type=pl.DeviceIdType.MESH,
            )
            rdma.start()
            ag_sends.append(rdma)

        for j in range(1, N_DEV):
            p = (my + j) % N_DEV
            recv_only(ag_recv_ref.at[j - 1], ag_recv_sems.at[j - 1]).wait_recv()
            out_ref[pl.ds(p * QROWS, QROWS), :] = (
                ag_recv_ref[j - 1].astype(jnp.float32))

        for qi in range(N_DEV):
            @pl.when(qi != my)
            def _():
                rs_rdma(qi).wait_send()
        for rdma in ag_sends:
            rdma.wait_send()

    out_flat = pl.pallas_call(
        body,
        out_shape=jax.ShapeDtypeStruct((T, D), jnp.float32),
        in_specs=[pl.BlockSpec(memory_space=pltpu.VMEM)] * 5,
        out_specs=pl.BlockSpec(memory_space=pltpu.VMEM),
        scratch_shapes=[
            pltpu.VMEM((N_DEV, QROWS, D), jnp.bfloat16),
            pltpu.VMEM((N_DEV - 1, QROWS, D), jnp.bfloat16),
            pltpu.VMEM((QROWS, D), jnp.bfloat16),
            pltpu.VMEM((N_DEV - 1, QROWS, D), jnp.bfloat16),
            pltpu.VMEM((SQ, HD_LOC), jnp.float32),
            pltpu.SemaphoreType.DMA((N_DEV,)),
            pltpu.SemaphoreType.DMA((N_DEV - 1,)),
            pltpu.SemaphoreType.DMA((N_DEV - 1,)),
            pltpu.SemaphoreType.DMA((N_DEV - 1,)),
        ],
        compiler_params=pltpu.CompilerParams(collective_id=0),
    )(x, Wq, Wo, Wk, Wv)
    return out_flat.reshape(B, SQ, D)


# device time: 24707 ns/iter; 1.9570x vs baseline; 1.1701x over previous
import os

import jax
import jax.numpy as jnp
from jax import lax
from jax.experimental import pallas as pl
from jax.experimental.pallas import tpu as pltpu

_HERE = os.path.dirname(os.path.abspath(__file__))
SKIP_COMM = os.path.exists(os.path.join(_HERE, ".skip_comm"))
SKIP_ATTN = os.path.exists(os.path.join(_HERE, ".skip_attn"))

N_DEV = 4
B, SQ, D = 2, 256, 768
T = B * SQ
QROWS = T // N_DEV
HQ_LOC, DH = 8, 64
HD_LOC = HQ_LOC * DH
KV_COLS = 2 * DH


def kernel(x, Wq, Wo, Wk, Wv):
    my_out = lax.axis_index("i")
    wqkv_host = jnp.concatenate(
        [(Wq * 0.125).astype(jnp.bfloat16),
         lax.dynamic_slice_in_dim(Wk, my_out * KV_COLS, KV_COLS, axis=1
                                  ).astype(jnp.bfloat16),
         lax.dynamic_slice_in_dim(Wv, my_out * KV_COLS, KV_COLS, axis=1
                                  ).astype(jnp.bfloat16)],
        axis=1,
    )
    wo_host = Wo.astype(jnp.bfloat16)
    x_host = x.astype(jnp.bfloat16)

    def body(x_ref, wqkv_ref, wo_ref, out_ref,
             stage_ref, rs_recv_ref,
             rs_send_sems, rs_recv_sems, ag_send_sems, ag_recv_sems):
        my = lax.axis_index("i")

        barrier_sem = pltpu.get_barrier_semaphore()
        for j in range(1, N_DEV):
            pl.semaphore_signal(
                barrier_sem, inc=1,
                device_id=((my + j) % N_DEV,),
                device_id_type=pl.DeviceIdType.MESH,
            )
        pl.semaphore_wait(barrier_sem, N_DEV - 1)

        wqkv = wqkv_ref[...]
        wo = wo_ref[...]

        def rs_rdma(q):
            return pltpu.make_async_remote_copy(
                src_ref=stage_ref.at[q],
                dst_ref=rs_recv_ref.at[(my - q) % N_DEV - 1],
                send_sem=rs_send_sems.at[q],
                recv_sem=rs_recv_sems.at[(my - q) % N_DEV - 1],
                device_id=(q,),
                device_id_type=pl.DeviceIdType.MESH,
            )

        for b in range(B):
            xb = x_ref[b]
            qkv = jnp.dot(xb, wqkv, preferred_element_type=jnp.float32
                          ).astype(jnp.bfloat16)
            q = qkv[:, :HD_LOC]
            k = qkv[:, HD_LOC:HD_LOC + KV_COLS]
            v = qkv[:, HD_LOC + KV_COLS:]

            if SKIP_ATTN:
                o = q
            else:
                o_heads = []
                for h in range(HQ_LOC):
                    qh = q[:, h * DH:(h + 1) * DH]
                    g = h // 4
                    kh = k[:, g * DH:(g + 1) * DH]
                    vh = v[:, g * DH:(g + 1) * DH]
                    s = lax.dot_general(
                        qh, kh, (((1,), (1,)), ((), ())),
                        preferred_element_type=jnp.float32,
                    )
                    p = jnp.exp(s)
                    l = jnp.sum(p, axis=1, keepdims=True)
                    o_heads.append(
                        jnp.dot(p.astype(jnp.bfloat16), vh,
                                preferred_element_type=jnp.float32) / l)
                o = jnp.concatenate(o_heads, axis=1).astype(jnp.bfloat16)

            partial = jnp.dot(o, wo,
                              preferred_element_type=jnp.float32)

            for half in range(2):
                qi = 2 * b + half
                stage_ref[qi] = partial[half * QROWS:(half + 1) * QROWS
                                        ].astype(jnp.bfloat16)

                if not SKIP_COMM:
                    @pl.when(qi != my)
                    def _():
                        rs_rdma(qi).start()

        if SKIP_COMM:
            for qi in range(N_DEV):
                out_ref[qi // 2, pl.ds((qi % 2) * QROWS, QROWS)] = stage_ref[qi]
            return

        def recv_only(dst_ref, recv_sem):
            return pltpu.make_async_remote_copy(
                src_ref=dst_ref, dst_ref=dst_ref,
                send_sem=rs_send_sems.at[0], recv_sem=recv_sem,
                device_id=(my,), device_id_type=pl.DeviceIdType.MESH,
            )

        for j in range(N_DEV - 1):
            recv_only(rs_recv_ref.at[j], rs_recv_sems.at[j]).wait_recv()
        red = stage_ref[my].astype(jnp.float32)
        for j in range(N_DEV - 1):
            red += rs_recv_ref[j].astype(jnp.float32)
        myrows = out_ref.at[my // 2, pl.ds((my % 2) * QROWS, QROWS), :]
        myrows[...] = red.astype(jnp.bfloat16)

        ag_sends = []
        for j in range(1, N_DEV):
            rdma = pltpu.make_async_remote_copy(
                src_ref=myrows,
                dst_ref=myrows,
                send_sem=ag_send_sems.at[j - 1],
                recv_sem=ag_recv_sems.at[N_DEV - 1 - j],
                device_id=((my + j) % N_DEV,),
                device_id_type=pl.DeviceIdType.MESH,
            )
            rdma.start()
            ag_sends.append(rdma)

        for j in range(1, N_DEV):
            p = (my + j) % N_DEV
            prows = out_ref.at[p // 2, pl.ds((p % 2) * QROWS, QROWS), :]
            recv_only(prows, ag_recv_sems.at[j - 1]).wait_recv()

        for qi in range(N_DEV):
            @pl.when(qi != my)
            def _():
                rs_rdma(qi).wait_send()
        for rdma in ag_sends:
            rdma.wait_send()

    return pl.pallas_call(
        body,
        out_shape=jax.ShapeDtypeStruct((B, SQ, D), jnp.bfloat16),
        in_specs=[pl.BlockSpec(memory_space=pltpu.VMEM)] * 3,
        out_specs=pl.BlockSpec(memory_space=pltpu.VMEM),
        scratch_shapes=[
            pltpu.VMEM((N_DEV, QROWS, D), jnp.bfloat16),
            pltpu.VMEM((N_DEV - 1, QROWS, D), jnp.bfloat16),
            pltpu.SemaphoreType.DMA((N_DEV,)),
            pltpu.SemaphoreType.DMA((N_DEV - 1,)),
            pltpu.SemaphoreType.DMA((N_DEV - 1,)),
            pltpu.SemaphoreType.DMA((N_DEV - 1,)),
        ],
        compiler_params=pltpu.CompilerParams(collective_id=0),
    )(x_host, wqkv_host, wo_host)


# device time: 22025 ns/iter; 2.1953x vs baseline; 1.1218x over previous
import os

import jax
import jax.numpy as jnp
from jax import lax
from jax.experimental import pallas as pl
from jax.experimental.pallas import tpu as pltpu

_HERE = os.path.dirname(os.path.abspath(__file__))
SKIP_COMM = os.path.exists(os.path.join(_HERE, ".skip_comm"))
SKIP_ATTN = os.path.exists(os.path.join(_HERE, ".skip_attn"))

N_DEV = 4
B, SQ, D = 2, 256, 768
T = B * SQ
BROWS = SQ // N_DEV
HQ_LOC, DH = 8, 64
HD_LOC = HQ_LOC * DH
KV_COLS = 2 * DH


def kernel(x, Wq, Wo, Wk, Wv):
    my_out = lax.axis_index("i")
    wqkv_host = jnp.concatenate(
        [Wq * 0.125,
         lax.dynamic_slice_in_dim(Wk, my_out * KV_COLS, KV_COLS, axis=1),
         lax.dynamic_slice_in_dim(Wv, my_out * KV_COLS, KV_COLS, axis=1)],
        axis=1,
    ).astype(jnp.bfloat16)
    wo_host = Wo.astype(jnp.bfloat16)
    x_host = x.astype(jnp.bfloat16)

    def body(x_ref, wqkv_ref, wo_ref, out_ref,
             stage_ref, rs_recv_ref,
             rs_send_sems, rs_recv_sems, ag_send_sems, ag_recv_sems):
        my = lax.axis_index("i")

        barrier_sem = pltpu.get_barrier_semaphore()
        for j in range(1, N_DEV):
            pl.semaphore_signal(
                barrier_sem, inc=1,
                device_id=((my + j) % N_DEV,),
                device_id_type=pl.DeviceIdType.MESH,
            )
        pl.semaphore_wait(barrier_sem, N_DEV - 1)

        wqkv = wqkv_ref[...]
        wo = wo_ref[...]

        def rs_rdma(b, blk):
            return pltpu.make_async_remote_copy(
                src_ref=stage_ref.at[b, blk],
                dst_ref=rs_recv_ref.at[b, (my - blk) % N_DEV - 1],
                send_sem=rs_send_sems.at[b, blk],
                recv_sem=rs_recv_sems.at[b, (my - blk) % N_DEV - 1],
                device_id=(blk,),
                device_id_type=pl.DeviceIdType.MESH,
            )

        for b in range(B):
            xb = x_ref[b]
            qkv = jnp.dot(xb, wqkv, preferred_element_type=jnp.float32
                          ).astype(jnp.bfloat16)
            q = qkv[:, :HD_LOC]
            k = qkv[:, HD_LOC:HD_LOC + KV_COLS]
            v = qkv[:, HD_LOC + KV_COLS:]

            if SKIP_ATTN:
                o = q
            else:
                o_heads = []
                for h in range(HQ_LOC):
                    qh = q[:, h * DH:(h + 1) * DH]
                    g = h // 4
                    kh = k[:, g * DH:(g + 1) * DH]
                    vh = v[:, g * DH:(g + 1) * DH]
                    s = lax.dot_general(
                        qh, kh, (((1,), (1,)), ((), ())),
                        preferred_element_type=jnp.float32,
                    )
                    p = jnp.exp(s)
                    l = jnp.sum(p, axis=1, keepdims=True)
                    o_heads.append(
                        jnp.dot(p.astype(jnp.bfloat16), vh,
                                preferred_element_type=jnp.float32) / l)
                o = jnp.concatenate(o_heads, axis=1).astype(jnp.bfloat16)

            partial = jnp.dot(o, wo,
                              preferred_element_type=jnp.float32)

            for blk in range(N_DEV):
                stage_ref[b, blk] = partial[blk * BROWS:(blk + 1) * BROWS
                                            ].astype(jnp.bfloat16)

                if not SKIP_COMM:
                    @pl.when(blk != my)
                    def _():
                        rs_rdma(b, blk).start()

        if SKIP_COMM:
            for b in range(B):
                for blk in range(N_DEV):
                    out_ref[pl.ds(b * SQ + blk * BROWS, BROWS), :] = (
                        stage_ref[b, blk])
            return

        def recv_only(dst_ref, recv_sem):
            return pltpu.make_async_remote_copy(
                src_ref=dst_ref, dst_ref=dst_ref,
                send_sem=rs_send_sems.at[0, 0], recv_sem=recv_sem,
                device_id=(my,), device_id_type=pl.DeviceIdType.MESH,
            )

        ag_sends = []
        for b in range(B):
            for j in range(N_DEV - 1):
                recv_only(rs_recv_ref.at[b, j],
                          rs_recv_sems.at[b, j]).wait_recv()
            red = stage_ref[b, my].astype(jnp.float32)
            for j in range(N_DEV - 1):
                red += rs_recv_ref[b, j].astype(jnp.float32)
            myrows = out_ref.at[pl.ds(b * SQ + my * BROWS, BROWS), :]
            myrows[...] = red.astype(jnp.bfloat16)

            for j in range(1, N_DEV):
                rdma = pltpu.make_async_remote_copy(
                    src_ref=myrows,
                    dst_ref=myrows,
                    send_sem=ag_send_sems.at[b, j - 1],
                    recv_sem=ag_recv_sems.at[b, N_DEV - 1 - j],
                    device_id=((my + j) % N_DEV,),
                    device_id_type=pl.DeviceIdType.MESH,
                )
                rdma.start()
                ag_sends.append(rdma)

        for b in range(B):
            for j in range(1, N_DEV):
                p = (my + j) % N_DEV
                prows = out_ref.at[pl.ds(b * SQ + p * BROWS, BROWS), :]
                recv_only(prows, ag_recv_sems.at[b, j - 1]).wait_recv()

        for b in range(B):
            for blk in range(N_DEV):
                @pl.when(blk != my)
                def _():
                    rs_rdma(b, blk).wait_send()
        for rdma in ag_sends:
            rdma.wait_send()

    out_flat = pl.pallas_call(
        body,
        out_shape=jax.ShapeDtypeStruct((T, D), jnp.bfloat16),
        in_specs=[pl.BlockSpec(memory_space=pltpu.VMEM)] * 3,
        out_specs=pl.BlockSpec(memory_space=pltpu.VMEM),
        scratch_shapes=[
            pltpu.VMEM((B, N_DEV, BROWS, D), jnp.bfloat16),
            pltpu.VMEM((B, N_DEV - 1, BROWS, D), jnp.bfloat16),
            pltpu.SemaphoreType.DMA((B, N_DEV)),
            pltpu.SemaphoreType.DMA((B, N_DEV - 1)),
            pltpu.SemaphoreType.DMA((B, N_DEV - 1)),
            pltpu.SemaphoreType.DMA((B, N_DEV - 1)),
        ],
        compiler_params=pltpu.CompilerParams(collective_id=0),
    )(x_host, wqkv_host, wo_host)
    return out_flat.reshape(B, SQ, D)
